# baseline (speedup 1.0000x reference)
import functools

import jax
import jax.numpy as jnp
from jax import lax
from jax.experimental import pallas as pl
from jax.experimental.pallas import tpu as pltpu
from jax.experimental.pallas import tpu_sc as plsc

VOCAB = 1000
BT = 1024 * 50
NC, NS, L = 2, 16, 16
NW = NC * NS
ROWS_W = 32
TROWS = 3 * NW + ROWS_W * NW
QUART = BT // 4
GUNROLL = 4
LCHUNK = 64
PER_W = BT // NW


def _lse_body(x_ref, o_ref):
    x = x_ref[...]
    m = jnp.max(x, axis=1)
    s = jnp.sum(jnp.exp(x - m[:, None]), axis=1)
    o_ref[...] = m + jnp.log(s)


def _row_lse(table_padded):
    return pl.pallas_call(
        _lse_body,
        out_shape=jax.ShapeDtypeStruct((VOCAB,), jnp.float32),
    )(table_padded)


def _loss_body(p_ref, o_ref):
    o_ref[0, 0] = jnp.sum(p_ref[...]) * (1.0 / BT)


def _loss_mean(parts):
    return pl.pallas_call(
        _loss_body,
        out_shape=jax.ShapeDtypeStruct((1, 1), jnp.float32),
        out_specs=pl.BlockSpec(memory_space=pltpu.MemorySpace.SMEM),
    )(parts)


_sc_mesh = plsc.VectorSubcoreMesh(core_axis_name="c", subcore_axis_name="s")


@functools.partial(
    pl.kernel,
    mesh=_sc_mesh,
    compiler_params=pltpu.CompilerParams(needs_layout_passes=False),
    out_type=[
        jax.ShapeDtypeStruct((VOCAB, BT), jnp.float32),
        jax.ShapeDtypeStruct((NW, L), jnp.float32),
    ],
    scratch_types=[
        pltpu.VMEM((BT,), jnp.int32),
        pltpu.VMEM((1, QUART), jnp.float32),
        pltpu.VMEM((1, QUART), jnp.float32),
        pltpu.VMEM((1, VOCAB), jnp.float32),
        pltpu.VMEM((1, VOCAB), jnp.float32),
        pltpu.VMEM((VOCAB,), jnp.float32),
        pltpu.VMEM((LCHUNK,), jnp.int32),
        pltpu.VMEM((LCHUNK,), jnp.int32),
        pltpu.VMEM((LCHUNK,), jnp.float32),
        pltpu.VMEM((L,), jnp.float32),
        pltpu.SemaphoreType.DMA,
        pltpu.SemaphoreType.DMA,
        pltpu.SemaphoreType.DMA,
        pltpu.SemaphoreType.DMA,
        pltpu.SemaphoreType.DMA,
    ],
)
def _sc_cols(idx_hbm, tgt_hbm, tt_hbm, tflat_hbm, lse_hbm,
             out_hbm, part_hbm,
             idx_v, qa_v, qb_v, ta_v, tb_v, lse_v, tgt_v, flat_v, pick_v,
             acc_v, tsa, tsb, wsa, wsb, lsem):
    wid = lax.axis_index("s") * NC + lax.axis_index("c")
    zeros = jnp.zeros((L,), jnp.int32)

    pltpu.async_copy(tt_hbm.at[pl.ds(wid, 1)], ta_v, tsa)
    pltpu.async_copy(tt_hbm.at[pl.ds(wid + NW, 1)], tb_v, tsb)
    pltpu.sync_copy(idx_hbm, idx_v)
    pltpu.sync_copy(lse_hbm, lse_v)

    def drain(dst_ref, sem):
        pltpu.make_async_copy(out_hbm.at[pl.ds(0, 1), pl.ds(0, QUART)],
                              dst_ref, sem).wait()

    def trow_wait(trow_ref, sem):
        pltpu.make_async_copy(tt_hbm.at[pl.ds(0, 1)], trow_ref, sem).wait()

    def gather_quarter(q, trow_ref, buf_ref):
        def body(jj, _):
            for u in range(GUNROLL):
                o = L * (GUNROLL * jj + u)
                vec = idx_v[pl.ds(q * QUART + o, L)]
                g = plsc.load_gather(trow_ref, [zeros, vec])
                buf_ref[0, pl.ds(o, L)] = g
            return 0
        lax.fori_loop(0, QUART // (L * GUNROLL), body, 0)

    def do_row(v, trow_ref, tsem, first):
        trow_wait(trow_ref, tsem)
        for q in range(4):
            buf, wsem = (qa_v, wsa) if q % 2 == 0 else (qb_v, wsb)
            if not (first and q < 2):
                drain(buf, wsem)
            gather_quarter(q, trow_ref, buf)
            pltpu.async_copy(
                buf, out_hbm.at[pl.ds(v, 1), pl.ds(q * QUART, QUART)], wsem)

    v0 = wid
    do_row(v0, ta_v, tsa, True)
    pltpu.async_copy(tt_hbm.at[pl.ds(v0 + 2 * NW, 1)], ta_v, tsa)
    do_row(v0 + NW, tb_v, tsb, False)
    pltpu.async_copy(tt_hbm.at[pl.ds(v0 + 3 * NW, 1)], tb_v, tsb)

    def k2body(k2, _):
        va = v0 + 2 * NW * k2

        def aseg():
            do_row(va, ta_v, tsa, False)
            pltpu.async_copy(tt_hbm.at[pl.ds(va + 2 * NW, 1)], ta_v, tsa)

        def bseg():
            do_row(va + NW, tb_v, tsb, False)
            pltpu.async_copy(tt_hbm.at[pl.ds(va + 3 * NW, 1)], tb_v, tsb)

        aseg()

        @pl.when(va + NW < VOCAB)
        def _():
            bseg()

        return 0

    lax.fori_loop(1, ROWS_W // 2, k2body, 0)

    drain(qa_v, wsa)
    drain(qb_v, wsb)
    trow_wait(ta_v, tsa)
    trow_wait(tb_v, tsb)

    base = wid * PER_W

    def loss_body(j, acc):
        off = base + j * LCHUNK
        pltpu.sync_copy(tgt_hbm.at[pl.ds(off, LCHUNK)], tgt_v)
        for k in range(LCHUNK // L):
            sl = pl.ds(k * L, L)
            flat_v[sl] = idx_v[pl.ds(off + k * L, L)] * VOCAB + tgt_v[sl]
        pltpu.async_copy(tflat_hbm.at[flat_v], pick_v, lsem).wait()
        for k in range(LCHUNK // L):
            sl = pl.ds(k * L, L)
            lg = plsc.load_gather(lse_v, [idx_v[pl.ds(off + k * L, L)]])
            acc = acc + (lg - pick_v[sl])
        return acc

    acc = lax.fori_loop(0, PER_W // LCHUNK, loss_body,
                        jnp.zeros((L,), jnp.float32))
    acc_v[...] = acc
    pltpu.sync_copy(acc_v, part_hbm.at[wid])


def kernel(idx, targets, table):
    idxf = idx.reshape(-1).astype(jnp.int32)
    tgtf = targets.reshape(-1).astype(jnp.int32)
    table_padded = jnp.pad(table, ((0, 0), (0, 24)),
                           constant_values=-jnp.inf)
    lse = _row_lse(table_padded)
    tt = jnp.pad(table.T, ((0, TROWS - VOCAB), (0, 0)))
    out_t, parts = _sc_cols(idxf, tgtf, tt, table.reshape(-1), lse)
    loss = _loss_mean(parts)[0, 0]
    return out_t.T, loss


# speedup vs baseline: 4.0131x; 4.0131x over previous
import functools

import jax
import jax.numpy as jnp
from jax import lax
from jax.experimental import pallas as pl
from jax.experimental.pallas import tpu as pltpu
from jax.experimental.pallas import tpu_sc as plsc

VOCAB = 1000
BT = 1024 * 50
NC, NS, L = 2, 16, 16
NW = NC * NS
ROWS_W = 32
TROWS = 3 * NW + ROWS_W * NW
QUART = BT // 4
GUNROLL = 8
LCHUNK = 64
PER_W = BT // NW


def _lse_body(x_ref, o_ref):
    x = x_ref[...]
    m = jnp.max(x, axis=1)
    s = jnp.sum(jnp.exp(x - m[:, None]), axis=1)
    o_ref[...] = m + jnp.log(s)


def _row_lse(table_padded):
    return pl.pallas_call(
        _lse_body,
        out_shape=jax.ShapeDtypeStruct((VOCAB,), jnp.float32),
    )(table_padded)


def _loss_body(p_ref, o_ref):
    o_ref[0, 0] = jnp.sum(p_ref[...]) * (1.0 / BT)


def _loss_mean(parts):
    return pl.pallas_call(
        _loss_body,
        out_shape=jax.ShapeDtypeStruct((1, 1), jnp.float32),
        out_specs=pl.BlockSpec(memory_space=pltpu.MemorySpace.SMEM),
    )(parts)


_sc_mesh = plsc.VectorSubcoreMesh(core_axis_name="c", subcore_axis_name="s")


@functools.partial(
    pl.kernel,
    mesh=_sc_mesh,
    compiler_params=pltpu.CompilerParams(needs_layout_passes=False),
    out_type=[
        jax.ShapeDtypeStruct((VOCAB, BT), jnp.float32),
        jax.ShapeDtypeStruct((NW, L), jnp.float32),
    ],
    scratch_types=[
        pltpu.VMEM((BT,), jnp.int32),
        pltpu.VMEM((1, QUART), jnp.float32),
        pltpu.VMEM((1, QUART), jnp.float32),
        pltpu.VMEM((1, VOCAB), jnp.float32),
        pltpu.VMEM((1, VOCAB), jnp.float32),
        pltpu.VMEM((VOCAB,), jnp.float32),
        pltpu.VMEM((LCHUNK,), jnp.int32),
        pltpu.VMEM((LCHUNK,), jnp.int32),
        pltpu.VMEM((LCHUNK,), jnp.float32),
        pltpu.VMEM((L,), jnp.float32),
        pltpu.SemaphoreType.DMA,
        pltpu.SemaphoreType.DMA,
        pltpu.SemaphoreType.DMA,
        pltpu.SemaphoreType.DMA,
        pltpu.SemaphoreType.DMA,
    ],
)
def _sc_cols(idx_hbm, tgt_hbm, tt_hbm, tflat_hbm, lse_hbm,
             out_hbm, part_hbm,
             idx_v, qa_v, qb_v, ta_v, tb_v, lse_v, tgt_v, flat_v, pick_v,
             acc_v, tsa, tsb, wsa, wsb, lsem):
    wid = lax.axis_index("s") * NC + lax.axis_index("c")
    zeros = jnp.zeros((L,), jnp.int32)

    pltpu.async_copy(tt_hbm.at[pl.ds(wid, 1)], ta_v, tsa)
    pltpu.async_copy(tt_hbm.at[pl.ds(wid + NW, 1)], tb_v, tsb)
    pltpu.sync_copy(idx_hbm, idx_v)
    pltpu.sync_copy(lse_hbm, lse_v)

    def drain(dst_ref, sem):
        pltpu.make_async_copy(out_hbm.at[pl.ds(0, 1), pl.ds(0, QUART)],
                              dst_ref, sem).wait()

    def trow_wait(trow_ref, sem):
        pltpu.make_async_copy(tt_hbm.at[pl.ds(0, 1)], trow_ref, sem).wait()

    def gather_quarter(q, trow_ref, buf_ref):
        @plsc.parallel_loop(0, QUART, L, unroll=GUNROLL)
        def body(o):
            vec = idx_v[pl.ds(q * QUART + o, L)]
            g = plsc.load_gather(trow_ref, [zeros, vec])
            buf_ref[0, pl.ds(o, L)] = g

    def do_row(v, trow_ref, tsem, first):
        trow_wait(trow_ref, tsem)
        for q in range(4):
            buf, wsem = (qa_v, wsa) if q % 2 == 0 else (qb_v, wsb)
            if not (first and q < 2):
                drain(buf, wsem)
            gather_quarter(q, trow_ref, buf)
            pltpu.async_copy(
                buf, out_hbm.at[pl.ds(v, 1), pl.ds(q * QUART, QUART)], wsem)

    v0 = wid
    do_row(v0, ta_v, tsa, True)
    pltpu.async_copy(tt_hbm.at[pl.ds(v0 + 2 * NW, 1)], ta_v, tsa)
    do_row(v0 + NW, tb_v, tsb, False)
    pltpu.async_copy(tt_hbm.at[pl.ds(v0 + 3 * NW, 1)], tb_v, tsb)

    def k2body(k2, _):
        va = v0 + 2 * NW * k2

        def aseg():
            do_row(va, ta_v, tsa, False)
            pltpu.async_copy(tt_hbm.at[pl.ds(va + 2 * NW, 1)], ta_v, tsa)

        def bseg():
            do_row(va + NW, tb_v, tsb, False)
            pltpu.async_copy(tt_hbm.at[pl.ds(va + 3 * NW, 1)], tb_v, tsb)

        aseg()

        @pl.when(va + NW < VOCAB)
        def _():
            bseg()

        return 0

    lax.fori_loop(1, ROWS_W // 2, k2body, 0)

    drain(qa_v, wsa)
    drain(qb_v, wsb)
    trow_wait(ta_v, tsa)
    trow_wait(tb_v, tsb)

    base = wid * PER_W

    def loss_body(j, acc):
        off = base + j * LCHUNK
        pltpu.sync_copy(tgt_hbm.at[pl.ds(off, LCHUNK)], tgt_v)
        for k in range(LCHUNK // L):
            sl = pl.ds(k * L, L)
            flat_v[sl] = idx_v[pl.ds(off + k * L, L)] * VOCAB + tgt_v[sl]
        pltpu.async_copy(tflat_hbm.at[flat_v], pick_v, lsem).wait()
        for k in range(LCHUNK // L):
            sl = pl.ds(k * L, L)
            lg = plsc.load_gather(lse_v, [idx_v[pl.ds(off + k * L, L)]])
            acc = acc + (lg - pick_v[sl])
        return acc

    acc = lax.fori_loop(0, PER_W // LCHUNK, loss_body,
                        jnp.zeros((L,), jnp.float32))
    acc_v[...] = acc
    pltpu.sync_copy(acc_v, part_hbm.at[wid])


def kernel(idx, targets, table):
    idxf = idx.reshape(-1).astype(jnp.int32)
    tgtf = targets.reshape(-1).astype(jnp.int32)
    table_padded = jnp.pad(table, ((0, 0), (0, 24)),
                           constant_values=-jnp.inf)
    lse = _row_lse(table_padded)
    tt = jnp.pad(table.T, ((0, TROWS - VOCAB), (0, 0)))
    out_t, parts = _sc_cols(idxf, tgtf, tt, table.reshape(-1), lse)
    loss = _loss_mean(parts)[0, 0]
    return out_t.T, loss


# speedup vs baseline: 4.3439x; 1.0824x over previous
import functools

import jax
import jax.numpy as jnp
from jax import lax
from jax.experimental import pallas as pl
from jax.experimental.pallas import tpu as pltpu
from jax.experimental.pallas import tpu_sc as plsc

VOCAB = 1000
BT = 1024 * 50
NC, NS, L = 2, 16, 16
NW = NC * NS
ROWS_W = 32
QUART = BT // 4
GUNROLL = 16
LCHUNK = 64
PER_W = BT // NW


def _lse_body(x_ref, o_ref):
    x = x_ref[...]
    m = jnp.max(x, axis=1)
    s = jnp.sum(jnp.exp(x - m[:, None]), axis=1)
    o_ref[...] = m + jnp.log(s)


def _row_lse(table):
    return pl.pallas_call(
        _lse_body,
        out_shape=jax.ShapeDtypeStruct((VOCAB,), jnp.float32),
    )(table)


def _loss_body(p_ref, o_ref):
    o_ref[0, 0] = jnp.sum(p_ref[...]) * (1.0 / BT)


def _loss_mean(parts):
    return pl.pallas_call(
        _loss_body,
        out_shape=jax.ShapeDtypeStruct((1, 1), jnp.float32),
        out_specs=pl.BlockSpec(memory_space=pltpu.MemorySpace.SMEM),
    )(parts)


_sc_mesh = plsc.VectorSubcoreMesh(core_axis_name="c", subcore_axis_name="s")


@functools.partial(
    pl.kernel,
    mesh=_sc_mesh,
    compiler_params=pltpu.CompilerParams(needs_layout_passes=False),
    out_type=[
        jax.ShapeDtypeStruct((VOCAB, BT), jnp.float32),
        jax.ShapeDtypeStruct((NW, L), jnp.float32),
    ],
    scratch_types=[
        pltpu.VMEM((BT,), jnp.int32),
        pltpu.VMEM((1, QUART), jnp.float32),
        pltpu.VMEM((1, QUART), jnp.float32),
        pltpu.VMEM((1, VOCAB), jnp.float32),
        pltpu.VMEM((1, VOCAB), jnp.float32),
        pltpu.VMEM((VOCAB,), jnp.float32),
        pltpu.VMEM((LCHUNK,), jnp.int32),
        pltpu.VMEM((LCHUNK,), jnp.int32),
        pltpu.VMEM((LCHUNK,), jnp.float32),
        pltpu.VMEM((L,), jnp.float32),
        pltpu.SemaphoreType.DMA,
        pltpu.SemaphoreType.DMA,
        pltpu.SemaphoreType.DMA,
        pltpu.SemaphoreType.DMA,
        pltpu.SemaphoreType.DMA,
    ],
)
def _sc_cols(idx_hbm, tgt_hbm, tt_hbm, tflat_hbm, lse_hbm,
             out_hbm, part_hbm,
             idx_v, qa_v, qb_v, ta_v, tb_v, lse_v, tgt_v, flat_v, pick_v,
             acc_v, tsa, tsb, wsa, wsb, lsem):
    wid = lax.axis_index("s") * NC + lax.axis_index("c")
    zeros = jnp.zeros((L,), jnp.int32)

    pltpu.async_copy(tt_hbm.at[pl.ds(wid, 1)], ta_v, tsa)
    pltpu.async_copy(tt_hbm.at[pl.ds(wid + NW, 1)], tb_v, tsb)
    pltpu.sync_copy(idx_hbm, idx_v)
    pltpu.sync_copy(lse_hbm, lse_v)

    def drain(dst_ref, sem):
        pltpu.make_async_copy(out_hbm.at[pl.ds(0, 1), pl.ds(0, QUART)],
                              dst_ref, sem).wait()

    def trow_wait(trow_ref, sem):
        pltpu.make_async_copy(tt_hbm.at[pl.ds(0, 1)], trow_ref, sem).wait()

    def gather_quarter(q, trow_ref, buf_ref):
        @plsc.parallel_loop(0, QUART, L, unroll=GUNROLL)
        def body(o):
            vec = idx_v[pl.ds(q * QUART + o, L)]
            g = plsc.load_gather(trow_ref, [zeros, vec])
            buf_ref[0, pl.ds(o, L)] = g

    def do_row(v, trow_ref, tsem, first):
        trow_wait(trow_ref, tsem)
        for q in range(4):
            buf, wsem = (qa_v, wsa) if q % 2 == 0 else (qb_v, wsb)
            if not (first and q < 2):
                drain(buf, wsem)
            gather_quarter(q, trow_ref, buf)
            pltpu.async_copy(
                buf, out_hbm.at[pl.ds(v, 1), pl.ds(q * QUART, QUART)], wsem)

    v0 = wid
    do_row(v0, ta_v, tsa, True)
    pltpu.async_copy(tt_hbm.at[pl.ds(v0 + 2 * NW, 1)], ta_v, tsa)
    do_row(v0 + NW, tb_v, tsb, False)
    pltpu.async_copy(tt_hbm.at[pl.ds(v0 + 3 * NW, 1)], tb_v, tsb)

    def k2body(k2, _):
        va = v0 + 2 * NW * k2

        def aseg():
            do_row(va, ta_v, tsa, False)
            nxt = jnp.minimum(va + 2 * NW, VOCAB - 1)
            pltpu.async_copy(tt_hbm.at[pl.ds(nxt, 1)], ta_v, tsa)

        def bseg():
            do_row(va + NW, tb_v, tsb, False)
            nxt = jnp.minimum(va + 3 * NW, VOCAB - 1)
            pltpu.async_copy(tt_hbm.at[pl.ds(nxt, 1)], tb_v, tsb)

        aseg()

        @pl.when(va + NW < VOCAB)
        def _():
            bseg()

        return 0

    lax.fori_loop(1, ROWS_W // 2, k2body, 0)

    drain(qa_v, wsa)
    drain(qb_v, wsb)
    trow_wait(ta_v, tsa)
    trow_wait(tb_v, tsb)

    base = wid * PER_W

    def loss_body(j, acc):
        off = base + j * LCHUNK
        pltpu.sync_copy(tgt_hbm.at[pl.ds(off, LCHUNK)], tgt_v)
        for k in range(LCHUNK // L):
            sl = pl.ds(k * L, L)
            flat_v[sl] = idx_v[pl.ds(off + k * L, L)] * VOCAB + tgt_v[sl]
        pltpu.async_copy(tflat_hbm.at[flat_v], pick_v, lsem).wait()
        for k in range(LCHUNK // L):
            sl = pl.ds(k * L, L)
            lg = plsc.load_gather(lse_v, [idx_v[pl.ds(off + k * L, L)]])
            acc = acc + (lg - pick_v[sl])
        return acc

    acc = lax.fori_loop(0, PER_W // LCHUNK, loss_body,
                        jnp.zeros((L,), jnp.float32))
    acc_v[...] = acc
    pltpu.sync_copy(acc_v, part_hbm.at[wid])


def kernel(idx, targets, table):
    idxf = idx.reshape(-1).astype(jnp.int32)
    tgtf = targets.reshape(-1).astype(jnp.int32)
    lse = _row_lse(table)
    out_t, parts = _sc_cols(idxf, tgtf, table.T, table.reshape(-1), lse)
    loss = _loss_mean(parts)[0, 0]
    return out_t.T, loss


# speedup vs baseline: 4.4365x; 1.0213x over previous
import functools

import jax
import jax.numpy as jnp
from jax import lax
from jax.experimental import pallas as pl
from jax.experimental.pallas import tpu as pltpu
from jax.experimental.pallas import tpu_sc as plsc

VOCAB = 1000
BT = 1024 * 50
NC, NS, L = 2, 16, 16
NW = NC * NS
ROWS_W = 32
QUART = BT // 4
GUNROLL = 8
LCHUNK = 64
PER_W = BT // NW


def _lse_body(x_ref, o_ref):
    x = x_ref[...]
    m = jnp.max(x, axis=1)
    s = jnp.sum(jnp.exp(x - m[:, None]), axis=1)
    o_ref[...] = m + jnp.log(s)


def _row_lse(table):
    return pl.pallas_call(
        _lse_body,
        out_shape=jax.ShapeDtypeStruct((VOCAB,), jnp.float32),
    )(table)


def _loss_body(p_ref, o_ref):
    o_ref[0, 0] = jnp.sum(p_ref[...]) * (1.0 / BT)


def _loss_mean(parts):
    return pl.pallas_call(
        _loss_body,
        out_shape=jax.ShapeDtypeStruct((1, 1), jnp.float32),
        out_specs=pl.BlockSpec(memory_space=pltpu.MemorySpace.SMEM),
    )(parts)


_sc_mesh = plsc.VectorSubcoreMesh(core_axis_name="c", subcore_axis_name="s")


@functools.partial(
    pl.kernel,
    mesh=_sc_mesh,
    compiler_params=pltpu.CompilerParams(needs_layout_passes=False),
    out_type=[
        jax.ShapeDtypeStruct((VOCAB, BT), jnp.float32),
        jax.ShapeDtypeStruct((NW, L), jnp.float32),
    ],
    scratch_types=[
        pltpu.VMEM((BT,), jnp.int32),
        pltpu.VMEM((1, QUART), jnp.float32),
        pltpu.VMEM((1, QUART), jnp.float32),
        pltpu.VMEM((1, QUART), jnp.float32),
        pltpu.VMEM((1, QUART), jnp.float32),
        pltpu.VMEM((1, VOCAB), jnp.float32),
        pltpu.VMEM((1, VOCAB), jnp.float32),
        pltpu.VMEM((VOCAB,), jnp.float32),
        pltpu.VMEM((LCHUNK,), jnp.int32),
        pltpu.VMEM((LCHUNK,), jnp.int32),
        pltpu.VMEM((LCHUNK,), jnp.float32),
        pltpu.VMEM((L,), jnp.float32),
        pltpu.SemaphoreType.DMA,
        pltpu.SemaphoreType.DMA,
        pltpu.SemaphoreType.DMA,
        pltpu.SemaphoreType.DMA,
        pltpu.SemaphoreType.DMA,
        pltpu.SemaphoreType.DMA,
        pltpu.SemaphoreType.DMA,
    ],
)
def _sc_cols(idx_hbm, tgt_hbm, tt_hbm, tflat_hbm, lse_hbm,
             out_hbm, part_hbm,
             idx_v, qa0_v, qa1_v, qb0_v, qb1_v, ta_v, tb_v, lse_v,
             tgt_v, flat_v, pick_v, acc_v,
             tsa, tsb, wsa0, wsa1, wsb0, wsb1, lsem):
    wid = lax.axis_index("s") * NC + lax.axis_index("c")
    zeros = jnp.zeros((L,), jnp.int32)

    pltpu.async_copy(tt_hbm.at[pl.ds(wid, 1)], ta_v, tsa)
    pltpu.async_copy(tt_hbm.at[pl.ds(wid + NW, 1)], tb_v, tsb)
    pltpu.sync_copy(idx_hbm, idx_v)
    pltpu.sync_copy(lse_hbm, lse_v)

    def drain(dst_ref, sem):
        pltpu.make_async_copy(out_hbm.at[pl.ds(0, 1), pl.ds(0, QUART)],
                              dst_ref, sem).wait()

    def trow_wait(trow_ref, sem):
        pltpu.make_async_copy(tt_hbm.at[pl.ds(0, 1)], trow_ref, sem).wait()

    def abufs(q):
        return (qa0_v, wsa0) if q % 2 == 0 else (qa1_v, wsa1)

    def bbufs(q):
        return (qb0_v, wsb0) if q % 2 == 0 else (qb1_v, wsb1)

    def fire(buf, v, q, wsem):
        pltpu.async_copy(
            buf, out_hbm.at[pl.ds(v, 1), pl.ds(q * QUART, QUART)], wsem)

    def do_pair(va, vb, first):
        trow_wait(ta_v, tsa)
        trow_wait(tb_v, tsb)
        for q in range(4):
            ba, sa = abufs(q)
            bb, sb = bbufs(q)
            if not (first and q < 2):
                drain(ba, sa)
                drain(bb, sb)

            @plsc.parallel_loop(0, QUART, L, unroll=GUNROLL)
            def body(o):
                vec = idx_v[pl.ds(q * QUART + o, L)]
                ba[0, pl.ds(o, L)] = plsc.load_gather(ta_v, [zeros, vec])
                bb[0, pl.ds(o, L)] = plsc.load_gather(tb_v, [zeros, vec])

            fire(ba, va, q, sa)
            fire(bb, vb, q, sb)

    def do_single(va):
        trow_wait(ta_v, tsa)
        trow_wait(tb_v, tsb)
        for q in range(4):
            ba, sa = abufs(q)
            drain(ba, sa)

            @plsc.parallel_loop(0, QUART, L, unroll=GUNROLL)
            def body(o):
                vec = idx_v[pl.ds(q * QUART + o, L)]
                ba[0, pl.ds(o, L)] = plsc.load_gather(ta_v, [zeros, vec])

            fire(ba, va, q, sa)

    def prefetch(k2next):
        ra = jnp.minimum(wid + 2 * NW * k2next, VOCAB - 1)
        rb = jnp.minimum(wid + 2 * NW * k2next + NW, VOCAB - 1)
        pltpu.async_copy(tt_hbm.at[pl.ds(ra, 1)], ta_v, tsa)
        pltpu.async_copy(tt_hbm.at[pl.ds(rb, 1)], tb_v, tsb)

    do_pair(wid, wid + NW, True)
    prefetch(1)

    def k2body(k2, _):
        va = wid + 2 * NW * k2
        do_pair(va, va + NW, False)
        prefetch(k2 + 1)
        return 0

    lax.fori_loop(1, ROWS_W // 2 - 1, k2body, 0)

    vlast = wid + 2 * NW * (ROWS_W // 2 - 1)

    @pl.when(vlast + NW < VOCAB)
    def _():
        do_pair(vlast, vlast + NW, False)

    @pl.when(vlast + NW >= VOCAB)
    def _():
        do_single(vlast)

    drain(qa0_v, wsa0)
    drain(qa1_v, wsa1)
    drain(qb0_v, wsb0)
    drain(qb1_v, wsb1)

    base = wid * PER_W

    def loss_body(j, acc):
        off = base + j * LCHUNK
        pltpu.sync_copy(tgt_hbm.at[pl.ds(off, LCHUNK)], tgt_v)
        for k in range(LCHUNK // L):
            sl = pl.ds(k * L, L)
            flat_v[sl] = idx_v[pl.ds(off + k * L, L)] * VOCAB + tgt_v[sl]
        pltpu.async_copy(tflat_hbm.at[flat_v], pick_v, lsem).wait()
        for k in range(LCHUNK // L):
            sl = pl.ds(k * L, L)
            lg = plsc.load_gather(lse_v, [idx_v[pl.ds(off + k * L, L)]])
            acc = acc + (lg - pick_v[sl])
        return acc

    acc = lax.fori_loop(0, PER_W // LCHUNK, loss_body,
                        jnp.zeros((L,), jnp.float32))
    acc_v[...] = acc
    pltpu.sync_copy(acc_v, part_hbm.at[wid])


def kernel(idx, targets, table):
    idxf = idx.reshape(-1).astype(jnp.int32)
    tgtf = targets.reshape(-1).astype(jnp.int32)
    lse = _row_lse(table)
    out_t, parts = _sc_cols(idxf, tgtf, table.T, table.reshape(-1), lse)
    loss = _loss_mean(parts)[0, 0]
    return out_t.T, loss


# speedup vs baseline: 5.0942x; 1.1483x over previous
import functools

import jax
import jax.numpy as jnp
from jax import lax
from jax.experimental import pallas as pl
from jax.experimental.pallas import tpu as pltpu
from jax.experimental.pallas import tpu_sc as plsc

VOCAB = 1000
BT = 1024 * 50
NC, NS, L = 2, 16, 16
NW = NC * NS
ROWS_W = 32
QUART = BT // 4
GUNROLL = 8
LCHUNK = 64
PER_W = BT // NW


def _lse_body(x_ref, o_ref):
    x = x_ref[...]
    m = jnp.max(x, axis=1)
    s = jnp.sum(jnp.exp(x - m[:, None]), axis=1)
    o_ref[...] = m + jnp.log(s)


def _row_lse(table):
    return pl.pallas_call(
        _lse_body,
        out_shape=jax.ShapeDtypeStruct((VOCAB,), jnp.float32),
    )(table)


def _loss_body(p_ref, o_ref):
    o_ref[0, 0] = jnp.sum(p_ref[...]) * (1.0 / BT)


def _loss_mean(parts):
    return pl.pallas_call(
        _loss_body,
        out_shape=jax.ShapeDtypeStruct((1, 1), jnp.float32),
        out_specs=pl.BlockSpec(memory_space=pltpu.MemorySpace.SMEM),
    )(parts)


_sc_mesh = plsc.VectorSubcoreMesh(core_axis_name="c", subcore_axis_name="s")


@functools.partial(
    pl.kernel,
    mesh=_sc_mesh,
    compiler_params=pltpu.CompilerParams(needs_layout_passes=False),
    out_type=[
        jax.ShapeDtypeStruct((VOCAB, BT), jnp.float32),
        jax.ShapeDtypeStruct((NW, L), jnp.float32),
    ],
    scratch_types=[
        pltpu.VMEM((BT,), jnp.int32),
        pltpu.VMEM((1, QUART), jnp.float32),
        pltpu.VMEM((1, QUART), jnp.float32),
        pltpu.VMEM((1, QUART), jnp.float32),
        pltpu.VMEM((1, QUART), jnp.float32),
        pltpu.VMEM((1, VOCAB), jnp.float32),
        pltpu.VMEM((1, VOCAB), jnp.float32),
        pltpu.VMEM((VOCAB,), jnp.float32),
        pltpu.VMEM((PER_W,), jnp.int32),
        pltpu.VMEM((PER_W,), jnp.int32),
        pltpu.VMEM((PER_W,), jnp.float32),
        pltpu.VMEM((L,), jnp.float32),
        pltpu.SemaphoreType.DMA,
        pltpu.SemaphoreType.DMA,
        pltpu.SemaphoreType.DMA,
        pltpu.SemaphoreType.DMA,
        pltpu.SemaphoreType.DMA,
        pltpu.SemaphoreType.DMA,
        pltpu.SemaphoreType.DMA,
    ],
)
def _sc_cols(idx_hbm, tgt_hbm, tt_hbm, tflat_hbm, lse_hbm,
             out_hbm, part_hbm,
             idx_v, qa0_v, qa1_v, qb0_v, qb1_v, ta_v, tb_v, lse_v,
             tgt_v, flat_v, pick_v, acc_v,
             tsa, tsb, wsa0, wsa1, wsb0, wsb1, lsem):
    wid = lax.axis_index("s") * NC + lax.axis_index("c")
    zeros = jnp.zeros((L,), jnp.int32)

    pltpu.async_copy(tt_hbm.at[pl.ds(wid, 1)], ta_v, tsa)
    pltpu.async_copy(tt_hbm.at[pl.ds(wid + NW, 1)], tb_v, tsb)
    pltpu.sync_copy(idx_hbm, idx_v)
    pltpu.sync_copy(lse_hbm, lse_v)

    base = wid * PER_W
    pltpu.sync_copy(tgt_hbm.at[pl.ds(base, PER_W)], tgt_v)

    @plsc.parallel_loop(0, PER_W, L, unroll=4)
    def _flat(o):
        flat_v[pl.ds(o, L)] = (idx_v[pl.ds(base + o, L)] * VOCAB
                               + tgt_v[pl.ds(o, L)])

    for t in range(0, PER_W, 128):
        sz = min(128, PER_W - t)
        pltpu.async_copy(tflat_hbm.at[flat_v.at[pl.ds(t, sz)]],
                         pick_v.at[pl.ds(t, sz)], lsem)

    def drain(dst_ref, sem):
        pltpu.make_async_copy(out_hbm.at[pl.ds(0, 1), pl.ds(0, QUART)],
                              dst_ref, sem).wait()

    def trow_wait(trow_ref, sem):
        pltpu.make_async_copy(tt_hbm.at[pl.ds(0, 1)], trow_ref, sem).wait()

    def abufs(q):
        return (qa0_v, wsa0) if q % 2 == 0 else (qa1_v, wsa1)

    def bbufs(q):
        return (qb0_v, wsb0) if q % 2 == 0 else (qb1_v, wsb1)

    def fire(buf, v, q, wsem):
        pltpu.async_copy(
            buf, out_hbm.at[pl.ds(v, 1), pl.ds(q * QUART, QUART)], wsem)

    def do_pair(va, vb, first):
        trow_wait(ta_v, tsa)
        trow_wait(tb_v, tsb)
        for q in range(4):
            ba, sa = abufs(q)
            bb, sb = bbufs(q)
            if not (first and q < 2):
                drain(ba, sa)
                drain(bb, sb)

            @plsc.parallel_loop(0, QUART, L, unroll=GUNROLL)
            def body(o):
                vec = idx_v[pl.ds(q * QUART + o, L)]
                ba[0, pl.ds(o, L)] = plsc.load_gather(ta_v, [zeros, vec])
                bb[0, pl.ds(o, L)] = plsc.load_gather(tb_v, [zeros, vec])

            fire(ba, va, q, sa)
            fire(bb, vb, q, sb)

    def do_single(va):
        trow_wait(ta_v, tsa)
        trow_wait(tb_v, tsb)
        for q in range(4):
            ba, sa = abufs(q)
            drain(ba, sa)

            @plsc.parallel_loop(0, QUART, L, unroll=GUNROLL)
            def body(o):
                vec = idx_v[pl.ds(q * QUART + o, L)]
                ba[0, pl.ds(o, L)] = plsc.load_gather(ta_v, [zeros, vec])

            fire(ba, va, q, sa)

    def prefetch(k2next):
        ra = jnp.minimum(wid + 2 * NW * k2next, VOCAB - 1)
        rb = jnp.minimum(wid + 2 * NW * k2next + NW, VOCAB - 1)
        pltpu.async_copy(tt_hbm.at[pl.ds(ra, 1)], ta_v, tsa)
        pltpu.async_copy(tt_hbm.at[pl.ds(rb, 1)], tb_v, tsb)

    do_pair(wid, wid + NW, True)
    prefetch(1)

    def k2body(k2, _):
        va = wid + 2 * NW * k2
        do_pair(va, va + NW, False)
        prefetch(k2 + 1)
        return 0

    lax.fori_loop(1, ROWS_W // 2 - 1, k2body, 0)

    vlast = wid + 2 * NW * (ROWS_W // 2 - 1)

    @pl.when(vlast + NW < VOCAB)
    def _():
        do_pair(vlast, vlast + NW, False)

    @pl.when(vlast + NW >= VOCAB)
    def _():
        do_single(vlast)

    drain(qa0_v, wsa0)
    drain(qa1_v, wsa1)
    drain(qb0_v, wsb0)
    drain(qb1_v, wsb1)

    pltpu.make_async_copy(tflat_hbm.at[flat_v], pick_v, lsem).wait()

    @plsc.parallel_loop(0, PER_W, L, unroll=4,
                        carry=jnp.zeros((L,), jnp.float32))
    def _acc(o, acc):
        lg = plsc.load_gather(lse_v, [idx_v[pl.ds(base + o, L)]])
        return acc + (lg - pick_v[pl.ds(o, L)])

    acc_v[...] = _acc
    pltpu.sync_copy(acc_v, part_hbm.at[wid])


def kernel(idx, targets, table):
    idxf = idx.reshape(-1).astype(jnp.int32)
    tgtf = targets.reshape(-1).astype(jnp.int32)
    lse = _row_lse(table)
    out_t, parts = _sc_cols(idxf, tgtf, table.T, table.reshape(-1), lse)
    loss = _loss_mean(parts)[0, 0]
    return out_t.T, loss


# speedup vs baseline: 5.1342x; 1.0078x over previous
import functools

import jax
import jax.numpy as jnp
from jax import lax
from jax.experimental import pallas as pl
from jax.experimental.pallas import tpu as pltpu
from jax.experimental.pallas import tpu_sc as plsc

VOCAB = 1000
VPAD = 1024
BT = 1024 * 50
NC, NS, L = 2, 16, 16
NW = NC * NS
ROWS_W = 32
QUART = BT // 4
GUNROLL = 8
LCHUNK = 64
PER_W = BT // NW


def _lse_body(x_ref, o_ref):
    x = x_ref[...]
    m = jnp.max(x, axis=1)
    s = jnp.sum(jnp.exp(x - m[:, None]), axis=1)
    o_ref[...] = m + jnp.log(s)


def _row_lse(table):
    return pl.pallas_call(
        _lse_body,
        out_shape=jax.ShapeDtypeStruct((VOCAB,), jnp.float32),
    )(table)


def _loss_body(p_ref, o_ref):
    o_ref[0, 0] = jnp.sum(p_ref[...]) * (1.0 / BT)


def _loss_mean(parts):
    return pl.pallas_call(
        _loss_body,
        out_shape=jax.ShapeDtypeStruct((1, 1), jnp.float32),
        out_specs=pl.BlockSpec(memory_space=pltpu.MemorySpace.SMEM),
    )(parts)


_sc_mesh = plsc.VectorSubcoreMesh(core_axis_name="c", subcore_axis_name="s")


@functools.partial(
    pl.kernel,
    mesh=_sc_mesh,
    compiler_params=pltpu.CompilerParams(needs_layout_passes=False),
    out_type=[
        jax.ShapeDtypeStruct((VOCAB, BT), jnp.float32),
        jax.ShapeDtypeStruct((NW, L), jnp.float32),
    ],
    scratch_types=[
        pltpu.VMEM((BT,), jnp.int32),
        pltpu.VMEM((1, QUART), jnp.float32),
        pltpu.VMEM((1, QUART), jnp.float32),
        pltpu.VMEM((1, QUART), jnp.float32),
        pltpu.VMEM((1, QUART), jnp.float32),
        pltpu.VMEM((VPAD,), jnp.float32),
        pltpu.VMEM((VPAD,), jnp.float32),
        pltpu.VMEM((VOCAB,), jnp.float32),
        pltpu.VMEM((PER_W,), jnp.int32),
        pltpu.VMEM((PER_W,), jnp.int32),
        pltpu.VMEM((PER_W,), jnp.float32),
        pltpu.VMEM((L,), jnp.float32),
        pltpu.SemaphoreType.DMA,
        pltpu.SemaphoreType.DMA,
        pltpu.SemaphoreType.DMA,
        pltpu.SemaphoreType.DMA,
        pltpu.SemaphoreType.DMA,
        pltpu.SemaphoreType.DMA,
        pltpu.SemaphoreType.DMA,
    ],
)
def _sc_cols(idx_hbm, tgt_hbm, ttf_hbm, lse_hbm,
             out_hbm, part_hbm,
             idx_v, qa0_v, qa1_v, qb0_v, qb1_v, ta_v, tb_v, lse_v,
             tgt_v, flat_v, pick_v, acc_v,
             tsa, tsb, wsa0, wsa1, wsb0, wsb1, lsem):
    wid = lax.axis_index("s") * NC + lax.axis_index("c")

    pltpu.async_copy(ttf_hbm.at[pl.ds(wid * VPAD, VPAD)], ta_v, tsa)
    pltpu.async_copy(ttf_hbm.at[pl.ds((wid + NW) * VPAD, VPAD)], tb_v, tsb)
    pltpu.sync_copy(idx_hbm, idx_v)
    pltpu.sync_copy(lse_hbm, lse_v)

    base = wid * PER_W
    pltpu.sync_copy(tgt_hbm.at[pl.ds(base, PER_W)], tgt_v)

    @plsc.parallel_loop(0, PER_W, L, unroll=4)
    def _flat(o):
        flat_v[pl.ds(o, L)] = (tgt_v[pl.ds(o, L)] * VPAD
                               + idx_v[pl.ds(base + o, L)])

    for t in range(0, PER_W, 128):
        sz = min(128, PER_W - t)
        pltpu.async_copy(ttf_hbm.at[flat_v.at[pl.ds(t, sz)]],
                         pick_v.at[pl.ds(t, sz)], lsem)

    def drain(dst_ref, sem):
        pltpu.make_async_copy(out_hbm.at[pl.ds(0, 1), pl.ds(0, QUART)],
                              dst_ref, sem).wait()

    def trow_wait(trow_ref, sem):
        pltpu.make_async_copy(ttf_hbm.at[pl.ds(0, VPAD)], trow_ref,
                              sem).wait()

    def abufs(q):
        return (qa0_v, wsa0) if q % 2 == 0 else (qa1_v, wsa1)

    def bbufs(q):
        return (qb0_v, wsb0) if q % 2 == 0 else (qb1_v, wsb1)

    def fire(buf, v, q, wsem):
        pltpu.async_copy(
            buf, out_hbm.at[pl.ds(v, 1), pl.ds(q * QUART, QUART)], wsem)

    def do_pair(va, vb, first):
        trow_wait(ta_v, tsa)
        trow_wait(tb_v, tsb)
        for q in range(4):
            ba, sa = abufs(q)
            bb, sb = bbufs(q)
            if not (first and q < 2):
                drain(ba, sa)
                drain(bb, sb)

            @plsc.parallel_loop(0, QUART, L, unroll=GUNROLL)
            def body(o):
                vec = idx_v[pl.ds(q * QUART + o, L)]
                ba[0, pl.ds(o, L)] = plsc.load_gather(ta_v, [vec])
                bb[0, pl.ds(o, L)] = plsc.load_gather(tb_v, [vec])

            fire(ba, va, q, sa)
            fire(bb, vb, q, sb)

    def do_single(va):
        trow_wait(ta_v, tsa)
        trow_wait(tb_v, tsb)
        for q in range(4):
            ba, sa = abufs(q)
            drain(ba, sa)

            @plsc.parallel_loop(0, QUART, L, unroll=GUNROLL)
            def body(o):
                vec = idx_v[pl.ds(q * QUART + o, L)]
                ba[0, pl.ds(o, L)] = plsc.load_gather(ta_v, [vec])

            fire(ba, va, q, sa)

    def prefetch(k2next):
        ra = jnp.minimum(wid + 2 * NW * k2next, VOCAB - 1)
        rb = jnp.minimum(wid + 2 * NW * k2next + NW, VOCAB - 1)
        pltpu.async_copy(ttf_hbm.at[pl.ds(ra * VPAD, VPAD)], ta_v, tsa)
        pltpu.async_copy(ttf_hbm.at[pl.ds(rb * VPAD, VPAD)], tb_v, tsb)

    do_pair(wid, wid + NW, True)
    prefetch(1)

    def k2body(k2, _):
        va = wid + 2 * NW * k2
        do_pair(va, va + NW, False)
        prefetch(k2 + 1)
        return 0

    lax.fori_loop(1, ROWS_W // 2 - 1, k2body, 0)

    vlast = wid + 2 * NW * (ROWS_W // 2 - 1)

    @pl.when(vlast + NW < VOCAB)
    def _():
        do_pair(vlast, vlast + NW, False)

    @pl.when(vlast + NW >= VOCAB)
    def _():
        do_single(vlast)

    drain(qa0_v, wsa0)
    drain(qa1_v, wsa1)
    drain(qb0_v, wsb0)
    drain(qb1_v, wsb1)

    pltpu.make_async_copy(ttf_hbm.at[flat_v], pick_v, lsem).wait()

    @plsc.parallel_loop(0, PER_W, L, unroll=4,
                        carry=jnp.zeros((L,), jnp.float32))
    def _acc(o, acc):
        lg = plsc.load_gather(lse_v, [idx_v[pl.ds(base + o, L)]])
        return acc + (lg - pick_v[pl.ds(o, L)])

    acc_v[...] = _acc
    pltpu.sync_copy(acc_v, part_hbm.at[wid])


def kernel(idx, targets, table):
    idxf = idx.reshape(-1).astype(jnp.int32)
    tgtf = targets.reshape(-1).astype(jnp.int32)
    lse = _row_lse(table)
    ttf = jnp.pad(table.T, ((0, 0), (0, VPAD - VOCAB))).reshape(-1)
    out_t, parts = _sc_cols(idxf, tgtf, ttf, lse)
    loss = _loss_mean(parts)[0, 0]
    return out_t.T, loss


# speedup vs baseline: 5.1632x; 1.0056x over previous
import functools

import jax
import jax.numpy as jnp
from jax import lax
from jax.experimental import pallas as pl
from jax.experimental.pallas import tpu as pltpu
from jax.experimental.pallas import tpu_sc as plsc

VOCAB = 1000
VPAD = 1024
BT = 1024 * 50
NC, NS, L = 2, 16, 16
NW = NC * NS
ROWS_W = 32
QUART = BT // 4
GUNROLL = 16
LCHUNK = 64
PER_W = BT // NW


def _lse_body(x_ref, o_ref):
    x = x_ref[...]
    m = jnp.max(x, axis=1)
    s = jnp.sum(jnp.exp(x - m[:, None]), axis=1)
    o_ref[...] = m + jnp.log(s)


def _row_lse(table):
    return pl.pallas_call(
        _lse_body,
        out_shape=jax.ShapeDtypeStruct((VOCAB,), jnp.float32),
    )(table)


def _loss_body(p_ref, o_ref):
    o_ref[0, 0] = jnp.sum(p_ref[...]) * (1.0 / BT)


def _loss_mean(parts):
    return pl.pallas_call(
        _loss_body,
        out_shape=jax.ShapeDtypeStruct((1, 1), jnp.float32),
        out_specs=pl.BlockSpec(memory_space=pltpu.MemorySpace.SMEM),
    )(parts)


_sc_mesh = plsc.VectorSubcoreMesh(core_axis_name="c", subcore_axis_name="s")


@functools.partial(
    pl.kernel,
    mesh=_sc_mesh,
    compiler_params=pltpu.CompilerParams(needs_layout_passes=False),
    out_type=[
        jax.ShapeDtypeStruct((VOCAB, BT), jnp.float32),
        jax.ShapeDtypeStruct((NW, L), jnp.float32),
    ],
    scratch_types=[
        pltpu.VMEM((BT,), jnp.int32),
        pltpu.VMEM((1, QUART), jnp.float32),
        pltpu.VMEM((1, QUART), jnp.float32),
        pltpu.VMEM((1, QUART), jnp.float32),
        pltpu.VMEM((1, QUART), jnp.float32),
        pltpu.VMEM((VPAD,), jnp.float32),
        pltpu.VMEM((VPAD,), jnp.float32),
        pltpu.VMEM((VOCAB,), jnp.float32),
        pltpu.VMEM((PER_W,), jnp.int32),
        pltpu.VMEM((PER_W,), jnp.int32),
        pltpu.VMEM((PER_W,), jnp.float32),
        pltpu.VMEM((L,), jnp.float32),
        pltpu.SemaphoreType.DMA,
        pltpu.SemaphoreType.DMA,
        pltpu.SemaphoreType.DMA,
        pltpu.SemaphoreType.DMA,
        pltpu.SemaphoreType.DMA,
        pltpu.SemaphoreType.DMA,
        pltpu.SemaphoreType.DMA,
    ],
)
def _sc_cols(idx_hbm, tgt_hbm, ttf_hbm, lse_hbm,
             out_hbm, part_hbm,
             idx_v, qa0_v, qa1_v, qb0_v, qb1_v, ta_v, tb_v, lse_v,
             tgt_v, flat_v, pick_v, acc_v,
             tsa, tsb, wsa0, wsa1, wsb0, wsb1, lsem):
    wid = lax.axis_index("s") * NC + lax.axis_index("c")

    pltpu.async_copy(ttf_hbm.at[pl.ds(wid * VPAD, VPAD)], ta_v, tsa)
    pltpu.async_copy(ttf_hbm.at[pl.ds((wid + NW) * VPAD, VPAD)], tb_v, tsb)
    pltpu.sync_copy(idx_hbm, idx_v)
    pltpu.sync_copy(lse_hbm, lse_v)

    base = wid * PER_W
    pltpu.sync_copy(tgt_hbm.at[pl.ds(base, PER_W)], tgt_v)

    @plsc.parallel_loop(0, PER_W, L, unroll=4)
    def _flat(o):
        flat_v[pl.ds(o, L)] = (tgt_v[pl.ds(o, L)] * VPAD
                               + idx_v[pl.ds(base + o, L)])

    for t in range(0, PER_W, 128):
        sz = min(128, PER_W - t)
        pltpu.async_copy(ttf_hbm.at[flat_v.at[pl.ds(t, sz)]],
                         pick_v.at[pl.ds(t, sz)], lsem)

    def drain(dst_ref, sem):
        pltpu.make_async_copy(out_hbm.at[pl.ds(0, 1), pl.ds(0, QUART)],
                              dst_ref, sem).wait()

    def trow_wait(trow_ref, sem):
        pltpu.make_async_copy(ttf_hbm.at[pl.ds(0, VPAD)], trow_ref,
                              sem).wait()

    def abufs(q):
        return (qa0_v, wsa0) if q % 2 == 0 else (qa1_v, wsa1)

    def bbufs(q):
        return (qb0_v, wsb0) if q % 2 == 0 else (qb1_v, wsb1)

    def fire(buf, v, q, wsem):
        pltpu.async_copy(
            buf, out_hbm.at[pl.ds(v, 1), pl.ds(q * QUART, QUART)], wsem)

    def do_pair(va, vb, first):
        trow_wait(ta_v, tsa)
        trow_wait(tb_v, tsb)
        for q in range(4):
            ba, sa = abufs(q)
            bb, sb = bbufs(q)
            if not (first and q < 2):
                drain(ba, sa)
                drain(bb, sb)

            @plsc.parallel_loop(0, QUART, L, unroll=GUNROLL)
            def body(o):
                vec = idx_v[pl.ds(q * QUART + o, L)]
                ba[0, pl.ds(o, L)] = plsc.load_gather(ta_v, [vec])
                bb[0, pl.ds(o, L)] = plsc.load_gather(tb_v, [vec])

            fire(ba, va, q, sa)
            fire(bb, vb, q, sb)

    def do_single(va):
        trow_wait(ta_v, tsa)
        trow_wait(tb_v, tsb)
        for q in range(4):
            ba, sa = abufs(q)
            drain(ba, sa)

            @plsc.parallel_loop(0, QUART, L, unroll=GUNROLL)
            def body(o):
                vec = idx_v[pl.ds(q * QUART + o, L)]
                ba[0, pl.ds(o, L)] = plsc.load_gather(ta_v, [vec])

            fire(ba, va, q, sa)

    def prefetch(k2next):
        ra = jnp.minimum(wid + 2 * NW * k2next, VOCAB - 1)
        rb = jnp.minimum(wid + 2 * NW * k2next + NW, VOCAB - 1)
        pltpu.async_copy(ttf_hbm.at[pl.ds(ra * VPAD, VPAD)], ta_v, tsa)
        pltpu.async_copy(ttf_hbm.at[pl.ds(rb * VPAD, VPAD)], tb_v, tsb)

    do_pair(wid, wid + NW, True)
    prefetch(1)

    def k2body(k2, _):
        va = wid + 2 * NW * k2
        do_pair(va, va + NW, False)
        prefetch(k2 + 1)
        return 0

    lax.fori_loop(1, ROWS_W // 2 - 1, k2body, 0)

    vlast = wid + 2 * NW * (ROWS_W // 2 - 1)

    @pl.when(vlast + NW < VOCAB)
    def _():
        do_pair(vlast, vlast + NW, False)

    @pl.when(vlast + NW >= VOCAB)
    def _():
        do_single(vlast)

    drain(qa0_v, wsa0)
    drain(qa1_v, wsa1)
    drain(qb0_v, wsb0)
    drain(qb1_v, wsb1)

    pltpu.make_async_copy(ttf_hbm.at[flat_v], pick_v, lsem).wait()

    @plsc.parallel_loop(0, PER_W, L, unroll=4,
                        carry=jnp.zeros((L,), jnp.float32))
    def _acc(o, acc):
        lg = plsc.load_gather(lse_v, [idx_v[pl.ds(base + o, L)]])
        return acc + (lg - pick_v[pl.ds(o, L)])

    acc_v[...] = _acc
    pltpu.sync_copy(acc_v, part_hbm.at[wid])


def kernel(idx, targets, table):
    idxf = idx.reshape(-1).astype(jnp.int32)
    tgtf = targets.reshape(-1).astype(jnp.int32)
    lse = _row_lse(table)
    ttf = jnp.pad(table.T, ((0, 0), (0, VPAD - VOCAB))).reshape(-1)
    out_t, parts = _sc_cols(idxf, tgtf, ttf, lse)
    loss = _loss_mean(parts)[0, 0]
    return out_t.T, loss
